# Initial kernel scaffold; baseline (speedup 1.0000x reference)
import jax
import jax.numpy as jnp
from jax.experimental import pallas as pl


def kernel(feature, edge_index, graph_ids, W1, b1, W2, b2):
    raise NotImplementedError("write your pallas kernel here")


# speedup vs baseline: 1.6651x; 1.6651x over previous
import functools

import jax
import jax.numpy as jnp
from jax import lax
from jax.experimental import pallas as pl
from jax.experimental.pallas import tpu as pltpu
from jax.experimental.pallas import tpu_sc as plsc

N = 10000
N_PAD = 10240
E = 320000
E_PAD = 327680
D = 128
G = 64
NC = 2
NS = 16
EPW = E_PAD // (NC * NS)
CHUNK = 128
NCHUNK = EPW // CHUNK
ROWS_PT = N_PAD // NS

@functools.cache
def _make_agg_sc():
  mesh = plsc.VectorSubcoreMesh(core_axis_name="c", subcore_axis_name="s")

  @functools.partial(
      pl.kernel,
      mesh=mesh,
      out_type=jax.ShapeDtypeStruct((NC, N_PAD, D), jnp.float32),
      scratch_types=[
          pltpu.VMEM((CHUNK,), jnp.int32),
          pltpu.VMEM((CHUNK,), jnp.int32),
          pltpu.VMEM((CHUNK, D), jnp.float32),
          pltpu.VMEM_SHARED((N_PAD, D), jnp.float32),
          pltpu.SemaphoreType.DMA,
      ],
  )
  def _agg_sc(h_hbm, src_hbm, dst_hbm, zeros_hbm, out_hbm,
              sidx, didx, rows, acc, sem):
    c = lax.axis_index("c")
    s = lax.axis_index("s")
    r0 = s * ROWS_PT
    pltpu.sync_copy(zeros_hbm.at[pl.ds(r0, ROWS_PT)], acc.at[pl.ds(r0, ROWS_PT)])
    plsc.subcore_barrier()

    base = (c * NS + s) * EPW

    def body(i, carry):
        b = pl.multiple_of(base + i * CHUNK, 8)
        pltpu.sync_copy(src_hbm.at[pl.ds(b, CHUNK)], sidx)
        pltpu.sync_copy(dst_hbm.at[pl.ds(b, CHUNK)], didx)
        pltpu.async_copy(h_hbm.at[sidx], rows, sem).wait()
        pltpu.sync_copy(rows, acc.at[didx], add=True)
        return carry

    lax.fori_loop(0, NCHUNK, body, 0)
    plsc.subcore_barrier()
    pltpu.sync_copy(acc.at[pl.ds(r0, ROWS_PT)], out_hbm.at[c, pl.ds(r0, ROWS_PT)])

  return _agg_sc


_BLK = 256
_NBLK = N_PAD // _BLK


def _norm_col(d0, d1):
    deg = d0[:, 0:1] + d1[:, 0:1]
    return lax.rsqrt(jnp.maximum(deg, 1.0))


def _prep1_body(feat_ref, do0_ref, do1_ref, w_ref, o_ref):
    feat = feat_ref[...]
    iota = lax.broadcasted_iota(jnp.int32, (_BLK, D), 1)
    h0 = jnp.zeros((_BLK, D), jnp.float32)
    for t in range(4):
        h0 = h0 + (feat[:, t][:, None] == iota).astype(jnp.float32)
    m = jnp.dot(h0, w_ref[...], preferred_element_type=jnp.float32)
    o_ref[...] = m * _norm_col(do0_ref[...], do1_ref[...])


def _prep1(feat_p, do0, do1, W1):
    return pl.pallas_call(
        _prep1_body,
        grid=(_NBLK,),
        in_specs=[
            pl.BlockSpec((_BLK, D), lambda i: (i, 0)),
            pl.BlockSpec((_BLK, D), lambda i: (i, 0)),
            pl.BlockSpec((_BLK, D), lambda i: (i, 0)),
            pl.BlockSpec((D, D), lambda i: (0, 0)),
        ],
        out_specs=pl.BlockSpec((_BLK, D), lambda i: (i, 0)),
        out_shape=jax.ShapeDtypeStruct((N_PAD, D), jnp.float32),
    )(feat_p, do0, do1, W1)


def _mid_body(a0_ref, a1_ref, di0_ref, di1_ref, do0_ref, do1_ref,
              b1_ref, w_ref, o_ref):
    agg = a0_ref[...] + a1_ref[...]
    nd = _norm_col(di0_ref[...], di1_ref[...])
    h = jnp.maximum(agg * nd + b1_ref[...], 0.0)
    m = jnp.dot(h, w_ref[...], preferred_element_type=jnp.float32)
    o_ref[...] = m * _norm_col(do0_ref[...], do1_ref[...])


def _mid(a0, a1, di0, di1, do0, do1, b1r, W2):
    return pl.pallas_call(
        _mid_body,
        grid=(_NBLK,),
        in_specs=[
            pl.BlockSpec((_BLK, D), lambda i: (i, 0)),
            pl.BlockSpec((_BLK, D), lambda i: (i, 0)),
            pl.BlockSpec((_BLK, D), lambda i: (i, 0)),
            pl.BlockSpec((_BLK, D), lambda i: (i, 0)),
            pl.BlockSpec((_BLK, D), lambda i: (i, 0)),
            pl.BlockSpec((_BLK, D), lambda i: (i, 0)),
            pl.BlockSpec((1, D), lambda i: (0, 0)),
            pl.BlockSpec((D, D), lambda i: (0, 0)),
        ],
        out_specs=pl.BlockSpec((_BLK, D), lambda i: (i, 0)),
        out_shape=jax.ShapeDtypeStruct((N_PAD, D), jnp.float32),
    )(a0, a1, di0, di1, do0, do1, b1r, W2)


def _final_body(a0_ref, a1_ref, di0_ref, di1_ref, b2_ref, gid_ref,
                pooled_ref, nsum_ref):
    i = pl.program_id(0)
    agg = a0_ref[...] + a1_ref[...]
    nd = _norm_col(di0_ref[...], di1_ref[...])
    h = agg * nd + b2_ref[...]
    row = lax.broadcasted_iota(jnp.int32, (_BLK, 1), 0) + i * _BLK
    h = jnp.where(row < N, h, 0.0)
    rn = jnp.sqrt(jnp.sum(h * h, axis=1, keepdims=True))
    rn = jnp.where(row < N, rn, 0.0)
    blocksum = jnp.sum(rn)
    gid = gid_ref[0, 0, :]
    onehot = (gid[:, None] == lax.broadcasted_iota(jnp.int32, (_BLK, G), 1)
              ).astype(jnp.float32)
    contrib = lax.dot_general(onehot, h, (((0,), (0,)), ((), ())),
                              preferred_element_type=jnp.float32)

    @pl.when(i == 0)
    def _():
        pooled_ref[...] = jnp.zeros_like(pooled_ref)
        nsum_ref[...] = jnp.zeros_like(nsum_ref)

    pooled_ref[...] += contrib
    nsum_ref[...] += blocksum


def _final(a0, a1, di0, di1, b2r, gids_p):
    return pl.pallas_call(
        _final_body,
        grid=(_NBLK,),
        in_specs=[
            pl.BlockSpec((_BLK, D), lambda i: (i, 0)),
            pl.BlockSpec((_BLK, D), lambda i: (i, 0)),
            pl.BlockSpec((_BLK, D), lambda i: (i, 0)),
            pl.BlockSpec((_BLK, D), lambda i: (i, 0)),
            pl.BlockSpec((1, D), lambda i: (0, 0)),
            pl.BlockSpec((1, 1, _BLK), lambda i: (i, 0, 0)),
        ],
        out_specs=[
            pl.BlockSpec((G, D), lambda i: (0, 0)),
            pl.BlockSpec((8, 128), lambda i: (0, 0)),
        ],
        out_shape=[
            jax.ShapeDtypeStruct((G, D), jnp.float32),
            jax.ShapeDtypeStruct((8, 128), jnp.float32),
        ],
    )(a0, a1, di0, di1, b2r, gids_p)


def kernel(feature, edge_index, graph_ids, W1, b1, W2, b2):
    src = edge_index[0]
    dst = edge_index[1]
    epad = jnp.full((E_PAD - E,), N_PAD - 1, jnp.int32)
    src_p = jnp.concatenate([src, epad])
    dst_p = jnp.concatenate([dst, epad])
    feat_p = jnp.pad(feature, ((0, N_PAD - N), (0, D - feature.shape[1])),
                     constant_values=D + 7)
    gids_p = jnp.pad(graph_ids, (0, N_PAD - N)).reshape(_NBLK, 1, _BLK)
    zD = jnp.zeros((N_PAD, D), jnp.float32)
    onesD = jnp.ones((N_PAD, D), jnp.float32)
    b1r = b1.reshape(1, D)
    b2r = b2.reshape(1, D)

    agg = _make_agg_sc()
    din = agg(onesD, src_p, dst_p, zD)
    dout = agg(onesD, dst_p, src_p, zD)
    do0, do1 = dout[0], dout[1]
    di0, di1 = din[0], din[1]

    h1 = _prep1(feat_p, do0, do1, W1)
    a1 = _make_agg_sc()(h1, src_p, dst_p, zD)
    h2 = _mid(a1[0], a1[1], di0, di1, do0, do1, b1r, W2)
    a2 = _make_agg_sc()(h2, src_p, dst_p, zD)
    pooled, nsum = _final(a2[0], a2[1], di0, di1, b2r, gids_p)

    factor = jnp.sqrt(jnp.float32(D)) * jnp.float32(N) / nsum[0, 0]
    return pooled * factor


# speedup vs baseline: 3.2510x; 1.9525x over previous
import functools

import jax
import jax.numpy as jnp
from jax import lax
from jax.experimental import pallas as pl
from jax.experimental.pallas import tpu as pltpu
from jax.experimental.pallas import tpu_sc as plsc

N = 10000
N_PAD = 10240
E = 320000
E_PAD = 327680
D = 128
G = 64
NC = 2
NS = 16
CHUNK = 128
EROWS = E_PAD // CHUNK
NCHUNK = EROWS // (NC * NS)
NCHUNK_DEG = EROWS // NS
ROWS_PT = N_PAD // NS
NB = 2
SB = 16
DEG_K = 8


@functools.cache
def _make_agg_sc():
  mesh = plsc.VectorSubcoreMesh(core_axis_name="c", subcore_axis_name="s")

  @functools.partial(
      pl.kernel,
      mesh=mesh,
      out_type=jax.ShapeDtypeStruct((NC, N_PAD, D), jnp.float32),
      scratch_types=[
          pltpu.VMEM((SB, CHUNK), jnp.int32),
          pltpu.VMEM((SB, CHUNK), jnp.int32),
          pltpu.VMEM((NB, CHUNK, D), jnp.float32),
          pltpu.VMEM_SHARED((N_PAD, D), jnp.float32),
      ] + [pltpu.SemaphoreType.DMA] * NB,
  )
  def _agg_sc(h_hbm, src_hbm, dst_hbm, zeros_hbm, out_hbm,
              sidx, didx, rows, acc, *sems):
    c = lax.axis_index("c")
    s = lax.axis_index("s")
    r0 = s * ROWS_PT
    tb = (c * NS + s) * NCHUNK
    pltpu.sync_copy(zeros_hbm.at[pl.ds(r0, ROWS_PT)], acc.at[pl.ds(r0, ROWS_PT)])
    plsc.subcore_barrier()

    def group(g, carry):
        gb = tb + g * SB
        pltpu.sync_copy(src_hbm.at[pl.ds(gb, SB)], sidx)
        pltpu.sync_copy(dst_hbm.at[pl.ds(gb, SB)], didx)
        for b in range(NB):
            pltpu.async_copy(h_hbm.at[sidx.at[b]], rows.at[b], sems[b])
        for k in range(SB):
            b = k % NB
            pltpu.make_async_copy(h_hbm.at[sidx.at[k]], rows.at[b],
                                  sems[b]).wait()
            pltpu.sync_copy(rows.at[b], acc.at[didx.at[k]], add=True)
            if k + NB < SB:
                pltpu.async_copy(h_hbm.at[sidx.at[k + NB]], rows.at[b],
                                 sems[b])
        return carry

    lax.fori_loop(0, NCHUNK // SB, group, 0)
    plsc.subcore_barrier()
    pltpu.sync_copy(acc.at[pl.ds(r0, ROWS_PT)], out_hbm.at[c, pl.ds(r0, ROWS_PT)])

  return _agg_sc


@functools.cache
def _make_deg_sc():
  mesh = plsc.VectorSubcoreMesh(core_axis_name="c", subcore_axis_name="s")

  @functools.partial(
      pl.kernel,
      mesh=mesh,
      out_type=jax.ShapeDtypeStruct((NC, N_PAD, D), jnp.float32),
      scratch_types=[
          pltpu.VMEM((DEG_K, CHUNK), jnp.int32),
          pltpu.VMEM((CHUNK, D), jnp.float32),
          pltpu.VMEM_SHARED((N_PAD, D), jnp.float32),
          pltpu.SemaphoreType.DMA,
      ],
  )
  def _deg_sc(idx_hbm, zeros_hbm, ones_hbm, out_hbm, idx, ones_v, acc, sem):
    c = lax.axis_index("c")
    s = lax.axis_index("s")
    r0 = s * ROWS_PT
    tb = c * EROWS + s * NCHUNK_DEG
    pltpu.sync_copy(ones_hbm.at[pl.ds(0, CHUNK)], ones_v)
    pltpu.sync_copy(zeros_hbm.at[pl.ds(r0, ROWS_PT)], acc.at[pl.ds(r0, ROWS_PT)])
    plsc.subcore_barrier()

    def group(g, carry):
        pltpu.sync_copy(idx_hbm.at[pl.ds(tb + g * DEG_K, DEG_K)], idx)
        for b in range(DEG_K):
            pltpu.async_copy(ones_v, acc.at[idx.at[b]], sem, add=True)
        for b in range(DEG_K):
            pltpu.make_async_copy(ones_v, acc.at[idx.at[b]], sem).wait()
        return carry

    lax.fori_loop(0, NCHUNK_DEG // DEG_K, group, 0)
    plsc.subcore_barrier()
    pltpu.sync_copy(acc.at[pl.ds(r0, ROWS_PT)], out_hbm.at[c, pl.ds(r0, ROWS_PT)])

  return _deg_sc


_BLK = 256
_NBLK = N_PAD // _BLK


def _norm_col(d):
    return lax.rsqrt(jnp.maximum(d[:, 0:1], 1.0))


def _prep1_body(feat_ref, do_ref, w_ref, o_ref):
    feat = feat_ref[...]
    iota = lax.broadcasted_iota(jnp.int32, (_BLK, D), 1)
    h0 = jnp.zeros((_BLK, D), jnp.float32)
    for t in range(4):
        h0 = h0 + (feat[:, t][:, None] == iota).astype(jnp.float32)
    m = jnp.dot(h0, w_ref[...], preferred_element_type=jnp.float32)
    o_ref[...] = m * _norm_col(do_ref[...])


def _prep1(feat_p, do, W1):
    return pl.pallas_call(
        _prep1_body,
        grid=(_NBLK,),
        in_specs=[
            pl.BlockSpec((_BLK, D), lambda i: (i, 0)),
            pl.BlockSpec((_BLK, D), lambda i: (i, 0)),
            pl.BlockSpec((D, D), lambda i: (0, 0)),
        ],
        out_specs=pl.BlockSpec((_BLK, D), lambda i: (i, 0)),
        out_shape=jax.ShapeDtypeStruct((N_PAD, D), jnp.float32),
    )(feat_p, do, W1)


def _mid_body(a0_ref, a1_ref, di_ref, do_ref, b1_ref, w_ref, o_ref):
    agg = a0_ref[...] + a1_ref[...]
    h = jnp.maximum(agg * _norm_col(di_ref[...]) + b1_ref[...], 0.0)
    m = jnp.dot(h, w_ref[...], preferred_element_type=jnp.float32)
    o_ref[...] = m * _norm_col(do_ref[...])


def _mid(a0, a1, di, do, b1r, W2):
    return pl.pallas_call(
        _mid_body,
        grid=(_NBLK,),
        in_specs=[
            pl.BlockSpec((_BLK, D), lambda i: (i, 0)),
            pl.BlockSpec((_BLK, D), lambda i: (i, 0)),
            pl.BlockSpec((_BLK, D), lambda i: (i, 0)),
            pl.BlockSpec((_BLK, D), lambda i: (i, 0)),
            pl.BlockSpec((1, D), lambda i: (0, 0)),
            pl.BlockSpec((D, D), lambda i: (0, 0)),
        ],
        out_specs=pl.BlockSpec((_BLK, D), lambda i: (i, 0)),
        out_shape=jax.ShapeDtypeStruct((N_PAD, D), jnp.float32),
    )(a0, a1, di, do, b1r, W2)


def _final_body(a0_ref, a1_ref, di_ref, b2_ref, gid_ref,
                pooled_ref, nsum_ref):
    i = pl.program_id(0)
    agg = a0_ref[...] + a1_ref[...]
    h = agg * _norm_col(di_ref[...]) + b2_ref[...]
    row = lax.broadcasted_iota(jnp.int32, (_BLK, 1), 0) + i * _BLK
    h = jnp.where(row < N, h, 0.0)
    rn = jnp.sqrt(jnp.sum(h * h, axis=1, keepdims=True))
    blocksum = jnp.sum(rn)
    gid = gid_ref[0, 0, :]
    onehot = (gid[:, None] == lax.broadcasted_iota(jnp.int32, (_BLK, G), 1)
              ).astype(jnp.float32)
    contrib = lax.dot_general(onehot, h, (((0,), (0,)), ((), ())),
                              preferred_element_type=jnp.float32)

    @pl.when(i == 0)
    def _():
        pooled_ref[...] = jnp.zeros_like(pooled_ref)
        nsum_ref[...] = jnp.zeros_like(nsum_ref)

    pooled_ref[...] += contrib
    nsum_ref[...] += blocksum


def _final(a0, a1, di, b2r, gids_p):
    return pl.pallas_call(
        _final_body,
        grid=(_NBLK,),
        in_specs=[
            pl.BlockSpec((_BLK, D), lambda i: (i, 0)),
            pl.BlockSpec((_BLK, D), lambda i: (i, 0)),
            pl.BlockSpec((_BLK, D), lambda i: (i, 0)),
            pl.BlockSpec((1, D), lambda i: (0, 0)),
            pl.BlockSpec((1, 1, _BLK), lambda i: (i, 0, 0)),
        ],
        out_specs=[
            pl.BlockSpec((G, D), lambda i: (0, 0)),
            pl.BlockSpec((8, 128), lambda i: (0, 0)),
        ],
        out_shape=[
            jax.ShapeDtypeStruct((G, D), jnp.float32),
            jax.ShapeDtypeStruct((8, 128), jnp.float32),
        ],
    )(a0, a1, di, b2r, gids_p)


def kernel(feature, edge_index, graph_ids, W1, b1, W2, b2):
    src = edge_index[0]
    dst = edge_index[1]
    epad = jnp.full((E_PAD - E,), N_PAD - 1, jnp.int32)
    src2 = jnp.concatenate([src, epad]).reshape(EROWS, CHUNK)
    dst2 = jnp.concatenate([dst, epad]).reshape(EROWS, CHUNK)
    idxcat = jnp.concatenate([src2, dst2], axis=0)
    feat_p = jnp.pad(feature, ((0, N_PAD - N), (0, D - feature.shape[1])),
                     constant_values=D + 7)
    gids_p = jnp.pad(graph_ids, (0, N_PAD - N)).reshape(_NBLK, 1, _BLK)
    zD = jnp.zeros((N_PAD, D), jnp.float32)
    onesD = jnp.ones((N_PAD, D), jnp.float32)
    b1r = b1.reshape(1, D)
    b2r = b2.reshape(1, D)

    degs = _make_deg_sc()(idxcat, zD, onesD)
    h1 = _prep1(feat_p, degs[0], W1)
    agg = _make_agg_sc()
    a1 = agg(h1, src2, dst2, zD)
    h2 = _mid(a1[0], a1[1], degs[1], degs[0], b1r, W2)
    a2 = agg(h2, src2, dst2, zD)
    pooled, nsum = _final(a2[0], a2[1], degs[1], b2r, gids_p)

    factor = jnp.sqrt(jnp.float32(D)) * jnp.float32(N) / nsum[0, 0]
    return pooled * factor


# speedup vs baseline: 7.7130x; 2.3725x over previous
import functools

import jax
import jax.numpy as jnp
from jax import lax
from jax.experimental import pallas as pl
from jax.experimental.pallas import tpu as pltpu
from jax.experimental.pallas import tpu_sc as plsc

N = 10000
N_PAD = 10240
E = 320000
E_PAD = 327680
D = 128
G = 64
NC = 2
NS = 16
CHUNK = 128
EROWS = E_PAD // CHUNK
NCHUNK = EROWS // (NC * NS)
NCHUNK_DEG = EROWS // NS
ROWS_PT = N_PAD // NS
NB = 2
SB = 16
DEG_K = 8


@functools.cache
def _make_agg_sc():
  mesh = plsc.VectorSubcoreMesh(core_axis_name="c", subcore_axis_name="s")

  @functools.partial(
      pl.kernel,
      mesh=mesh,
      out_type=jax.ShapeDtypeStruct((NC, N_PAD, D), jnp.float32),
      scratch_types=[
          pltpu.VMEM((SB, CHUNK), jnp.int32),
          pltpu.VMEM((SB, CHUNK), jnp.int32),
          pltpu.VMEM((NB, CHUNK, D), jnp.float32),
          pltpu.VMEM_SHARED((N_PAD, D), jnp.float32),
      ] + [pltpu.SemaphoreType.DMA] * NB,
  )
  def _agg_sc(h_hbm, src_hbm, dst_hbm, zeros_hbm, out_hbm,
              sidx, didx, rows, acc, *sems):
    c = lax.axis_index("c")
    s = lax.axis_index("s")
    r0 = s * ROWS_PT
    tb = (c * NS + s) * NCHUNK
    pltpu.sync_copy(zeros_hbm.at[pl.ds(r0, ROWS_PT)], acc.at[pl.ds(r0, ROWS_PT)])
    plsc.subcore_barrier()

    def group(g, carry):
        gb = tb + g * SB
        pltpu.sync_copy(src_hbm.at[pl.ds(gb, SB)], sidx)
        pltpu.sync_copy(dst_hbm.at[pl.ds(gb, SB)], didx)
        for b in range(NB):
            pltpu.async_copy(h_hbm.at[sidx.at[b]], rows.at[b], sems[b])
        for k in range(SB):
            b = k % NB
            pltpu.make_async_copy(h_hbm.at[sidx.at[k]], rows.at[b],
                                  sems[b]).wait()
            pltpu.sync_copy(rows.at[b], acc.at[didx.at[k]], add=True)
            if k + NB < SB:
                pltpu.async_copy(h_hbm.at[sidx.at[k + NB]], rows.at[b],
                                 sems[b])
        return carry

    lax.fori_loop(0, NCHUNK // SB, group, 0)
    plsc.subcore_barrier()
    pltpu.sync_copy(acc.at[pl.ds(r0, ROWS_PT)], out_hbm.at[c, pl.ds(r0, ROWS_PT)])

  return _agg_sc


@functools.cache
def _make_deg_sc():
  mesh = plsc.VectorSubcoreMesh(core_axis_name="c", subcore_axis_name="s")

  @functools.partial(
      pl.kernel,
      mesh=mesh,
      out_type=jax.ShapeDtypeStruct((NC, N_PAD, D), jnp.float32),
      scratch_types=[
          pltpu.VMEM((DEG_K, CHUNK), jnp.int32),
          pltpu.VMEM((CHUNK, D), jnp.float32),
          pltpu.VMEM_SHARED((N_PAD, D), jnp.float32),
          pltpu.SemaphoreType.DMA,
      ],
  )
  def _deg_sc(idx_hbm, zeros_hbm, ones_hbm, out_hbm, idx, ones_v, acc, sem):
    c = lax.axis_index("c")
    s = lax.axis_index("s")
    r0 = s * ROWS_PT
    tb = c * EROWS + s * NCHUNK_DEG
    pltpu.sync_copy(ones_hbm.at[pl.ds(0, CHUNK)], ones_v)
    pltpu.sync_copy(zeros_hbm.at[pl.ds(r0, ROWS_PT)], acc.at[pl.ds(r0, ROWS_PT)])
    plsc.subcore_barrier()

    def group(g, carry):
        pltpu.sync_copy(idx_hbm.at[pl.ds(tb + g * DEG_K, DEG_K)], idx)
        for b in range(DEG_K):
            pltpu.async_copy(ones_v, acc.at[idx.at[b]], sem, add=True)
        for b in range(DEG_K):
            pltpu.make_async_copy(ones_v, acc.at[idx.at[b]], sem).wait()
        return carry

    lax.fori_loop(0, NCHUNK_DEG // DEG_K, group, 0)
    plsc.subcore_barrier()
    pltpu.sync_copy(acc.at[pl.ds(r0, ROWS_PT)], out_hbm.at[c, pl.ds(r0, ROWS_PT)])

  return _deg_sc


_BLK = 256
_NBLK = N_PAD // _BLK


def _norm_col(d):
    return lax.rsqrt(jnp.maximum(d[:, 0:1], 1.0))


def _prep1_body(feat_ref, do_ref, w_ref, o_ref):
    feat = feat_ref[...]
    iota = lax.broadcasted_iota(jnp.int32, (_BLK, D), 1)
    h0 = jnp.zeros((_BLK, D), jnp.float32)
    for t in range(4):
        h0 = h0 + (feat[:, t][:, None] == iota).astype(jnp.float32)
    m = jnp.dot(h0, w_ref[...], preferred_element_type=jnp.float32)
    o_ref[...] = m * _norm_col(do_ref[...])


def _prep1(feat_p, do, W1):
    return pl.pallas_call(
        _prep1_body,
        grid=(_NBLK,),
        in_specs=[
            pl.BlockSpec((_BLK, D), lambda i: (i, 0)),
            pl.BlockSpec((_BLK, D), lambda i: (i, 0)),
            pl.BlockSpec((D, D), lambda i: (0, 0)),
        ],
        out_specs=pl.BlockSpec((_BLK, D), lambda i: (i, 0)),
        out_shape=jax.ShapeDtypeStruct((N_PAD, D), jnp.float32),
    )(feat_p, do, W1)


def _mid_body(a0_ref, a1_ref, di_ref, do_ref, b1_ref, w_ref, o_ref):
    agg = a0_ref[...] + a1_ref[...]
    h = jnp.maximum(agg * _norm_col(di_ref[...]) + b1_ref[...], 0.0)
    m = jnp.dot(h, w_ref[...], preferred_element_type=jnp.float32)
    o_ref[...] = m * _norm_col(do_ref[...])


def _mid(a0, a1, di, do, b1r, W2):
    return pl.pallas_call(
        _mid_body,
        grid=(_NBLK,),
        in_specs=[
            pl.BlockSpec((_BLK, D), lambda i: (i, 0)),
            pl.BlockSpec((_BLK, D), lambda i: (i, 0)),
            pl.BlockSpec((_BLK, D), lambda i: (i, 0)),
            pl.BlockSpec((_BLK, D), lambda i: (i, 0)),
            pl.BlockSpec((1, D), lambda i: (0, 0)),
            pl.BlockSpec((D, D), lambda i: (0, 0)),
        ],
        out_specs=pl.BlockSpec((_BLK, D), lambda i: (i, 0)),
        out_shape=jax.ShapeDtypeStruct((N_PAD, D), jnp.float32),
    )(a0, a1, di, do, b1r, W2)


def _final_body(a0_ref, a1_ref, di_ref, b2_ref, gid_ref,
                pooled_ref, nsum_ref):
    i = pl.program_id(0)
    agg = a0_ref[...] + a1_ref[...]
    h = agg * _norm_col(di_ref[...]) + b2_ref[...]
    row = lax.broadcasted_iota(jnp.int32, (_BLK, 1), 0) + i * _BLK
    h = jnp.where(row < N, h, 0.0)
    rn = jnp.sqrt(jnp.sum(h * h, axis=1, keepdims=True))
    blocksum = jnp.sum(rn)
    gid = gid_ref[0, 0, :]
    onehot = (gid[:, None] == lax.broadcasted_iota(jnp.int32, (_BLK, G), 1)
              ).astype(jnp.float32)
    contrib = lax.dot_general(onehot, h, (((0,), (0,)), ((), ())),
                              preferred_element_type=jnp.float32)

    @pl.when(i == 0)
    def _():
        pooled_ref[...] = jnp.zeros_like(pooled_ref)
        nsum_ref[...] = jnp.zeros_like(nsum_ref)

    pooled_ref[...] += contrib
    nsum_ref[...] += blocksum


def _final(a0, a1, di, b2r, gids_p):
    return pl.pallas_call(
        _final_body,
        grid=(_NBLK,),
        in_specs=[
            pl.BlockSpec((_BLK, D), lambda i: (i, 0)),
            pl.BlockSpec((_BLK, D), lambda i: (i, 0)),
            pl.BlockSpec((_BLK, D), lambda i: (i, 0)),
            pl.BlockSpec((1, D), lambda i: (0, 0)),
            pl.BlockSpec((1, 1, _BLK), lambda i: (i, 0, 0)),
        ],
        out_specs=[
            pl.BlockSpec((G, D), lambda i: (0, 0)),
            pl.BlockSpec((8, 128), lambda i: (0, 0)),
        ],
        out_shape=[
            jax.ShapeDtypeStruct((G, D), jnp.float32),
            jax.ShapeDtypeStruct((8, 128), jnp.float32),
        ],
    )(a0, a1, di, b2r, gids_p)


def kernel(feature, edge_index, graph_ids, W1, b1, W2, b2):
    src = edge_index[0]
    dst = edge_index[1]
    epad = N + jnp.arange(E_PAD - E, dtype=jnp.int32) % (N_PAD - N)
    src2 = jnp.concatenate([src, epad]).reshape(EROWS, CHUNK)
    dst2 = jnp.concatenate([dst, epad]).reshape(EROWS, CHUNK)
    idxcat = jnp.concatenate([src2, dst2], axis=0)
    feat_p = jnp.pad(feature, ((0, N_PAD - N), (0, D - feature.shape[1])),
                     constant_values=D + 7)
    gids_p = jnp.pad(graph_ids, (0, N_PAD - N)).reshape(_NBLK, 1, _BLK)
    zD = jnp.zeros((N_PAD, D), jnp.float32)
    onesD = jnp.ones((N_PAD, D), jnp.float32)
    b1r = b1.reshape(1, D)
    b2r = b2.reshape(1, D)

    degs = _make_deg_sc()(idxcat, zD, onesD)
    h1 = _prep1(feat_p, degs[0], W1)
    agg = _make_agg_sc()
    a1 = agg(h1, src2, dst2, zD)
    h2 = _mid(a1[0], a1[1], degs[1], degs[0], b1r, W2)
    a2 = agg(h2, src2, dst2, zD)
    pooled, nsum = _final(a2[0], a2[1], degs[1], b2r, gids_p)

    factor = jnp.sqrt(jnp.float32(D)) * jnp.float32(N) / nsum[0, 0]
    return pooled * factor


# speedup vs baseline: 8.0881x; 1.0486x over previous
import functools

import jax
import jax.numpy as jnp
from jax import lax
from jax.experimental import pallas as pl
from jax.experimental.pallas import tpu as pltpu
from jax.experimental.pallas import tpu_sc as plsc

N = 10000
N_PAD = 10240
E = 320000
E_PAD = 327680
D = 128
G = 64
NC = 2
NS = 16
CHUNK = 128
EROWS = E_PAD // CHUNK
NCHUNK = EROWS // (NC * NS)
NCHUNK_DEG = EROWS // NS
ROWS_PT = N_PAD // NS
NB = 2
SB = 16
DEG_K = 8


@functools.cache
def _make_agg_sc():
  mesh = plsc.VectorSubcoreMesh(core_axis_name="c", subcore_axis_name="s")

  @functools.partial(
      pl.kernel,
      mesh=mesh,
      out_type=jax.ShapeDtypeStruct((NC, N_PAD, D), jnp.float32),
      scratch_types=[
          pltpu.VMEM((SB, CHUNK), jnp.int32),
          pltpu.VMEM((SB, CHUNK), jnp.int32),
          pltpu.VMEM((NB, CHUNK, D), jnp.float32),
          pltpu.VMEM_SHARED((N_PAD, D), jnp.float32),
      ] + [pltpu.SemaphoreType.DMA] * NB,
  )
  def _agg_sc(h_hbm, src_hbm, dst_hbm, zeros_hbm, out_hbm,
              sidx, didx, rows, acc, *sems):
    c = lax.axis_index("c")
    s = lax.axis_index("s")
    r0 = s * ROWS_PT
    tb = (c * NS + s) * NCHUNK
    pltpu.sync_copy(zeros_hbm.at[pl.ds(r0, ROWS_PT)], acc.at[pl.ds(r0, ROWS_PT)])
    plsc.subcore_barrier()

    def group(g, carry):
        gb = tb + g * SB
        pltpu.sync_copy(src_hbm.at[pl.ds(gb, SB)], sidx)
        pltpu.sync_copy(dst_hbm.at[pl.ds(gb, SB)], didx)
        for b in range(NB):
            pltpu.async_copy(h_hbm.at[sidx.at[b]], rows.at[b], sems[b])
        for k in range(SB):
            b = k % NB
            pltpu.make_async_copy(h_hbm.at[sidx.at[k]], rows.at[b],
                                  sems[b]).wait()
            pltpu.sync_copy(rows.at[b], acc.at[didx.at[k]], add=True)
            if k + NB < SB:
                pltpu.async_copy(h_hbm.at[sidx.at[k + NB]], rows.at[b],
                                 sems[b])
        return carry

    lax.fori_loop(0, NCHUNK // SB, group, 0)
    plsc.subcore_barrier()
    pltpu.sync_copy(acc.at[pl.ds(r0, ROWS_PT)], out_hbm.at[c, pl.ds(r0, ROWS_PT)])

  return _agg_sc


@functools.cache
def _make_deg_sc():
  mesh = plsc.VectorSubcoreMesh(core_axis_name="c", subcore_axis_name="s")

  @functools.partial(
      pl.kernel,
      mesh=mesh,
      out_type=jax.ShapeDtypeStruct((NC, N_PAD, D), jnp.float32),
      scratch_types=[
          pltpu.VMEM((DEG_K, CHUNK), jnp.int32),
          pltpu.VMEM((CHUNK, D), jnp.float32),
          pltpu.VMEM_SHARED((N_PAD, D), jnp.float32),
          pltpu.SemaphoreType.DMA,
      ],
  )
  def _deg_sc(idx_hbm, zeros_hbm, ones_hbm, out_hbm, idx, ones_v, acc, sem):
    c = lax.axis_index("c")
    s = lax.axis_index("s")
    r0 = s * ROWS_PT
    tb = c * EROWS + s * NCHUNK_DEG
    pltpu.sync_copy(ones_hbm.at[pl.ds(0, CHUNK)], ones_v)
    pltpu.sync_copy(zeros_hbm.at[pl.ds(r0, ROWS_PT)], acc.at[pl.ds(r0, ROWS_PT)])
    plsc.subcore_barrier()

    def group(g, carry):
        pltpu.sync_copy(idx_hbm.at[pl.ds(tb + g * DEG_K, DEG_K)], idx)
        for b in range(DEG_K):
            pltpu.async_copy(ones_v, acc.at[idx.at[b]], sem, add=True)
        for b in range(DEG_K):
            pltpu.make_async_copy(ones_v, acc.at[idx.at[b]], sem).wait()
        return carry

    lax.fori_loop(0, NCHUNK_DEG // DEG_K, group, 0)
    plsc.subcore_barrier()
    pltpu.sync_copy(acc.at[pl.ds(r0, ROWS_PT)], out_hbm.at[c, pl.ds(r0, ROWS_PT)])

  return _deg_sc


_BLK = 256
_NBLK = N_PAD // _BLK


def _norm_col(d):
    return lax.rsqrt(jnp.maximum(d[:, 0:1], 1.0))


def _prep1_body(feat_ref, d_ref, w_ref, o_ref):
    feat = feat_ref[...]
    iota = lax.broadcasted_iota(jnp.int32, (_BLK, D), 1)
    h0 = jnp.zeros((_BLK, D), jnp.float32)
    for t in range(4):
        h0 = h0 + (feat[:, t][:, None] == iota).astype(jnp.float32)
    m = jnp.dot(h0, w_ref[...], preferred_element_type=jnp.float32)
    o_ref[...] = m * _norm_col(d_ref[0])


def _prep1(feat_p, degs, W1):
    return pl.pallas_call(
        _prep1_body,
        grid=(_NBLK,),
        in_specs=[
            pl.BlockSpec((_BLK, D), lambda i: (i, 0)),
            pl.BlockSpec((1, _BLK, D), lambda i: (0, i, 0)),
            pl.BlockSpec((D, D), lambda i: (0, 0)),
        ],
        out_specs=pl.BlockSpec((_BLK, D), lambda i: (i, 0)),
        out_shape=jax.ShapeDtypeStruct((N_PAD, D), jnp.float32),
    )(feat_p, degs, W1)


def _mid_body(a_ref, d_ref, b1_ref, w_ref, o_ref):
    agg = a_ref[0] + a_ref[1]
    h = jnp.maximum(agg * _norm_col(d_ref[1]) + b1_ref[...], 0.0)
    m = jnp.dot(h, w_ref[...], preferred_element_type=jnp.float32)
    o_ref[...] = m * _norm_col(d_ref[0])


def _mid(a1, degs, b1r, W2):
    return pl.pallas_call(
        _mid_body,
        grid=(_NBLK,),
        in_specs=[
            pl.BlockSpec((2, _BLK, D), lambda i: (0, i, 0)),
            pl.BlockSpec((2, _BLK, D), lambda i: (0, i, 0)),
            pl.BlockSpec((1, D), lambda i: (0, 0)),
            pl.BlockSpec((D, D), lambda i: (0, 0)),
        ],
        out_specs=pl.BlockSpec((_BLK, D), lambda i: (i, 0)),
        out_shape=jax.ShapeDtypeStruct((N_PAD, D), jnp.float32),
    )(a1, degs, b1r, W2)


def _final_body(a_ref, di_ref, b2_ref, gid_ref,
                pooled_ref, nsum_ref):
    i = pl.program_id(0)
    agg = a_ref[0] + a_ref[1]
    h = agg * _norm_col(di_ref[0]) + b2_ref[...]
    row = lax.broadcasted_iota(jnp.int32, (_BLK, 1), 0) + i * _BLK
    h = jnp.where(row < N, h, 0.0)
    rn = jnp.sqrt(jnp.sum(h * h, axis=1, keepdims=True))
    blocksum = jnp.sum(rn)
    gid = gid_ref[0, 0, :]
    onehot = (gid[:, None] == lax.broadcasted_iota(jnp.int32, (_BLK, G), 1)
              ).astype(jnp.float32)
    contrib = lax.dot_general(onehot, h, (((0,), (0,)), ((), ())),
                              preferred_element_type=jnp.float32)

    @pl.when(i == 0)
    def _():
        pooled_ref[...] = jnp.zeros_like(pooled_ref)
        nsum_ref[...] = jnp.zeros_like(nsum_ref)

    pooled_ref[...] += contrib
    nsum_ref[...] += blocksum


def _final(a2, degs, b2r, gids_p):
    return pl.pallas_call(
        _final_body,
        grid=(_NBLK,),
        in_specs=[
            pl.BlockSpec((2, _BLK, D), lambda i: (0, i, 0)),
            pl.BlockSpec((1, _BLK, D), lambda i: (1, i, 0)),
            pl.BlockSpec((1, D), lambda i: (0, 0)),
            pl.BlockSpec((1, 1, _BLK), lambda i: (i, 0, 0)),
        ],
        out_specs=[
            pl.BlockSpec((G, D), lambda i: (0, 0)),
            pl.BlockSpec((8, 128), lambda i: (0, 0)),
        ],
        out_shape=[
            jax.ShapeDtypeStruct((G, D), jnp.float32),
            jax.ShapeDtypeStruct((8, 128), jnp.float32),
        ],
    )(a2, degs, b2r, gids_p)


def kernel(feature, edge_index, graph_ids, W1, b1, W2, b2):
    src = edge_index[0]
    dst = edge_index[1]
    epad = N + jnp.arange(E_PAD - E, dtype=jnp.int32) % (N_PAD - N)
    src2 = jnp.concatenate([src, epad]).reshape(EROWS, CHUNK)
    dst2 = jnp.concatenate([dst, epad]).reshape(EROWS, CHUNK)
    idxcat = jnp.concatenate([src2, dst2], axis=0)
    feat_p = jnp.pad(feature, ((0, N_PAD - N), (0, D - feature.shape[1])),
                     constant_values=D + 7)
    gids_p = jnp.pad(graph_ids, (0, N_PAD - N)).reshape(_NBLK, 1, _BLK)
    zD = jnp.zeros((N_PAD, D), jnp.float32)
    onesD = jnp.ones((N_PAD, D), jnp.float32)
    b1r = b1.reshape(1, D)
    b2r = b2.reshape(1, D)

    degs = _make_deg_sc()(idxcat, zD, onesD)
    h1 = _prep1(feat_p, degs, W1)
    agg = _make_agg_sc()
    a1 = agg(h1, src2, dst2, zD)
    h2 = _mid(a1, degs, b1r, W2)
    a2 = agg(h2, src2, dst2, zD)
    pooled, nsum = _final(a2, degs, b2r, gids_p)

    factor = jnp.sqrt(jnp.float32(D)) * jnp.float32(N) / nsum[0, 0]
    return pooled * factor
